# Initial kernel scaffold; baseline (speedup 1.0000x reference)
import jax
import jax.numpy as jnp
from jax.experimental import pallas as pl


def kernel(delta, v_old, G_idx):
    raise NotImplementedError("write your pallas kernel here")


# speedup vs baseline: 5.2005x; 5.2005x over previous
import functools

import jax
import jax.numpy as jnp
from jax import lax
from jax.experimental import pallas as pl
from jax.experimental.pallas import tpu as pltpu
from jax.experimental.pallas import tpu_sc as plsc

L = 16


@functools.lru_cache(maxsize=None)
def _build_sc_kernel(N, D):
    info = plsc.get_sparse_core_info()
    NC, NS = info.num_cores, info.num_subcores
    NW = NC * NS
    assert N % NW == 0
    rows_per_w = N // NW
    R = 256
    assert rows_per_w % R == 0
    nblocks = rows_per_w // R
    KD = D // L

    mesh = plsc.VectorSubcoreMesh(core_axis_name="c", subcore_axis_name="s")

    @functools.partial(
        pl.kernel,
        out_type=jax.ShapeDtypeStruct((N, D), jnp.float32),
        mesh=mesh,
        scratch_types=[
            pltpu.VMEM((R, D), jnp.float32),
            pltpu.VMEM((R,), jnp.int32),
            pltpu.VMEM((R, D), jnp.float32),
            pltpu.VMEM((2, D), jnp.float32),
        ],
    )
    def body(delta_hbm, vold_hbm, gidx_hbm, out_hbm, delta_v, idx_v, out_v, vold_v):
        wid = lax.axis_index("s") * NC + lax.axis_index("c")
        row0 = wid * rows_per_w

        pltpu.sync_copy(vold_hbm, vold_v)
        pw = [vold_v[0, pl.ds(k * L, L)] - vold_v[1, pl.ds(k * L, L)]
              for k in range(KD)]
        nw = [-p for p in pw]

        def block_body(g, carry):
            base = row0 + g * R
            pltpu.sync_copy(gidx_hbm.at[pl.ds(base, R)], idx_v)
            pltpu.sync_copy(delta_hbm.at[pl.ds(base, R)], delta_v)

            def group_body(gr, c):
                gbase = gr * L
                gv = idx_v[pl.ds(gbase, L)]
                sgnv = jnp.where(gv == 0, jnp.float32(1.0), jnp.float32(-1.0))
                for r in range(L):
                    row = gbase + r
                    sfv = jnp.full((L,), sgnv[r])
                    for k in range(KD):
                        dv = delta_v[row, pl.ds(k * L, L)]
                        out_v[row, pl.ds(k * L, L)] = dv * pw[k] * sfv
                return c

            lax.fori_loop(0, R // L, group_body, 0)
            pltpu.sync_copy(out_v, out_hbm.at[pl.ds(base, R)])
            return carry

        lax.fori_loop(0, nblocks, block_body, 0)

    return body


def kernel(delta, v_old, G_idx):
    N, D = delta.shape
    return _build_sc_kernel(N, D)(delta, v_old, G_idx.astype(jnp.int32))
